# baseline (speedup 1.0000x reference)
import functools

import jax
import jax.numpy as jnp
from jax import lax
from jax.experimental import pallas as pl
from jax.experimental.pallas import tpu as pltpu
from jax.experimental.pallas import tpu_sc as plsc

N = 51200
E = 819200
B = 16
T = 50
NPG = 64
H = 64
GCN_OUT = 4 * H
RNN_H = 2 * GCN_OUT
NUM_CLASSES = 10

NS = 16
NSTRIPE = N // NS
CH = 128
KCH = 8
KS = 4
EROWS = E // CH
NROWS = N // CH

_SC_MESH = plsc.VectorSubcoreMesh(core_axis_name="c", subcore_axis_name="s")
_SC_PARAMS = pltpu.CompilerParams(use_tc_tiling_on_sc=False)


def _sc_prep_body(feat2, emb2, dst2, zeros8, ones8,
                  x0_out, deg_out,
                  featv, gidx, rows, dsti, onesv, dacc, gsem, ssem):
    c = lax.axis_index("c")
    s = lax.axis_index("s")

    rows_per_sub = NSTRIPE // CH

    def chunk_a(i, carry):
        pltpu.sync_copy(feat2.at[s * rows_per_sub + i], featv)
        for k in range(CH // 16):
            v = featv[pl.ds(k * 16, 16)]
            gidx[pl.ds(k * 16, 16)] = v + c * 257
        pltpu.async_copy(emb2.at[gidx], rows, gsem).wait()
        pltpu.sync_copy(rows, x0_out.at[c, pl.ds(s * NSTRIPE + i * CH, CH)])
        return carry

    lax.fori_loop(0, rows_per_sub, chunk_a, 0)

    pltpu.sync_copy(zeros8, dacc.at[pl.ds(s * NSTRIPE, NSTRIPE)])
    pltpu.sync_copy(ones8, onesv)
    plsc.subcore_barrier()

    w = s * 2 + c
    rows_per_w = EROWS // 32

    def chunk_b(i, carry):
        r0 = w * rows_per_w + i * KCH
        pltpu.sync_copy(dst2.at[pl.ds(r0, KCH)], dsti)
        hs = [pltpu.async_copy(onesv, dacc.at[dsti.at[j]], ssem, add=True)
              for j in range(KCH)]
        for h in hs:
            h.wait()
        return carry

    lax.fori_loop(0, rows_per_w // KCH, chunk_b, 0)
    plsc.subcore_barrier()
    pltpu.sync_copy(dacc.at[pl.ds(s * NSTRIPE, NSTRIPE)],
                    deg_out.at[c, pl.ds(s * NSTRIPE, NSTRIPE)])


@functools.partial(jax.jit)
def _sc_prep(feat2, emb2, dst2, zeros8, ones8):
    return pl.kernel(
        _sc_prep_body,
        out_type=(jax.ShapeDtypeStruct((2, N, 32), jnp.float32),
                  jax.ShapeDtypeStruct((2, N, 8), jnp.float32)),
        mesh=_SC_MESH,
        scratch_types=[
            pltpu.VMEM((CH,), jnp.int32),
            pltpu.VMEM((CH,), jnp.int32),
            pltpu.VMEM((CH, 32), jnp.float32),
            pltpu.VMEM((KCH, CH), jnp.int32),
            pltpu.VMEM((CH, 8), jnp.float32),
            pltpu.VMEM_SHARED((N, 8), jnp.float32),
            pltpu.SemaphoreType.DMA,
            pltpu.SemaphoreType.DMA,
        ],
        compiler_params=_SC_PARAMS,
    )(feat2, emb2, dst2, zeros8, ones8)


CS = 64
KSB = 4
ESROWS = E // CS


def _sc_segsum_body(x2, src2, dst2, zeros32,
                    s_out,
                    srci, dsti, gidx, rows, acc, gsem, ssem):
    c = lax.axis_index("c")
    s = lax.axis_index("s")

    pltpu.sync_copy(zeros32, acc.at[pl.ds(s * NSTRIPE, NSTRIPE)])
    plsc.subcore_barrier()

    cbase = c * N
    rows_per_sub = ESROWS // NS
    nblocks = rows_per_sub // KSB
    base_r = s * rows_per_sub

    def load_idx(b, r0):
        pltpu.sync_copy(src2.at[pl.ds(r0, KSB)], srci.at[b])
        pltpu.sync_copy(dst2.at[pl.ds(r0, KSB)], dsti.at[b])

    def fire_g(b):
        for j in range(KSB):
            for k in range(CS // 16):
                v = srci[b, j, pl.ds(k * 16, 16)]
                gidx[b, j, pl.ds(k * 16, 16)] = v + cbase
        for j in range(KSB):
            pltpu.async_copy(x2.at[gidx.at[b, j]], rows.at[b, j], gsem)

    def wait_g(b):
        for j in range(KSB):
            pltpu.make_async_copy(x2.at[gidx.at[b, j]], rows.at[b, j], gsem).wait()

    def fire_s(b):
        for j in range(KSB):
            pltpu.async_copy(rows.at[b, j], acc.at[dsti.at[b, j]], ssem, add=True)

    def wait_s(b):
        for j in range(KSB):
            pltpu.make_async_copy(rows.at[b, j], acc.at[dsti.at[b, j]], ssem).wait()

    load_idx(0, base_r)
    fire_g(0)

    def blockpair(G, carry):
        for b in range(2):
            load_idx(1 - b, base_r + (2 * G + b + 1) * KSB)
            wait_g(b)
            fire_s(b)
            fire_g(1 - b)
            wait_s(b)
        return carry

    lax.fori_loop(0, nblocks // 2 - 1, blockpair, 0)

    load_idx(1, base_r + (nblocks - 1) * KSB)
    wait_g(0)
    fire_s(0)
    fire_g(1)
    wait_s(0)
    wait_g(1)
    fire_s(1)
    wait_s(1)

    plsc.subcore_barrier()
    pltpu.sync_copy(acc.at[pl.ds(s * NSTRIPE, NSTRIPE)],
                    s_out.at[c, pl.ds(s * NSTRIPE, NSTRIPE)])


@functools.partial(jax.jit)
def _sc_segsum(x2, src2, dst2, zeros32):
    return pl.kernel(
        _sc_segsum_body,
        out_type=jax.ShapeDtypeStruct((2, N, 32), jnp.float32),
        mesh=_SC_MESH,
        scratch_types=[
            pltpu.VMEM((2, KSB, CS), jnp.int32),
            pltpu.VMEM((2, KSB, CS), jnp.int32),
            pltpu.VMEM((2, KSB, CS), jnp.int32),
            pltpu.VMEM((2, KSB, CS, 32), jnp.float32),
            pltpu.VMEM_SHARED((N, 32), jnp.float32),
            pltpu.SemaphoreType.DMA,
            pltpu.SemaphoreType.DMA,
        ],
        compiler_params=_SC_PARAMS,
    )(x2, src2, dst2, zeros32)


_BS = 2048
_GRID1 = N // _BS


def _k1_body(xr, sr, dr, wsr, wnr, br, ar, tout, mout, acc):
    i = pl.program_id(0)
    x64 = jnp.concatenate([xr[0], xr[1]], axis=1)
    s64 = jnp.concatenate([sr[0], sr[1]], axis=1)
    deg = jnp.maximum(dr[0][:, 0:1] + dr[1][:, 0:1], 1.0)
    hn = s64 / deg
    t = (jnp.dot(x64, wsr[...], preferred_element_type=jnp.float32)
         + jnp.dot(hn, wnr[...], preferred_element_type=jnp.float32)
         + br[...])
    t = jnp.where(t >= 0, t, ar[...] * t)
    tout[...] = t

    @pl.when(i == 0)
    def _():
        acc[...] = jnp.zeros_like(acc)

    acc[0:1, :] += jnp.sum(t, axis=0, keepdims=True)
    acc[1:2, :] += jnp.sum(t * t, axis=0, keepdims=True)

    @pl.when(i == _GRID1 - 1)
    def _():
        mout[...] = acc[...]


def _k1(x, s, degc, wsT, wnT, b, a):
    return pl.pallas_call(
        _k1_body,
        grid=(_GRID1,),
        in_specs=[
            pl.BlockSpec((2, _BS, 32), lambda i: (0, i, 0)),
            pl.BlockSpec((2, _BS, 32), lambda i: (0, i, 0)),
            pl.BlockSpec((2, _BS, 8), lambda i: (0, i, 0)),
            pl.BlockSpec((64, 64), lambda i: (0, 0)),
            pl.BlockSpec((64, 64), lambda i: (0, 0)),
            pl.BlockSpec((1, 64), lambda i: (0, 0)),
            pl.BlockSpec((1, 64), lambda i: (0, 0)),
        ],
        out_specs=[
            pl.BlockSpec((_BS, 64), lambda i: (i, 0)),
            pl.BlockSpec((2, 64), lambda i: (0, 0)),
        ],
        out_shape=[
            jax.ShapeDtypeStruct((N, 64), jnp.float32),
            jax.ShapeDtypeStruct((2, 64), jnp.float32),
        ],
        scratch_shapes=[pltpu.VMEM((2, 64), jnp.float32)],
    )(x, s, degc, wsT, wnT, b, a)


def _k2_body(tr, momr, gr, betr, gmr, xout, pout):
    m = momr[0:1, :] * (1.0 / N)
    v = momr[1:2, :] * (1.0 / N) - m * m
    sc = gr[...] * lax.rsqrt(v + 1e-5)
    xn = (tr[...] - m) * sc + betr[...]
    xout[...] = jnp.stack([xn[:, :32], xn[:, 32:]], axis=0)
    pout[...] = jnp.dot(gmr[...], xn, preferred_element_type=jnp.float32)


def _k2(t, mom, g, beta, gmat):
    return pl.pallas_call(
        _k2_body,
        grid=(_GRID1,),
        in_specs=[
            pl.BlockSpec((_BS, 64), lambda i: (i, 0)),
            pl.BlockSpec((2, 64), lambda i: (0, 0)),
            pl.BlockSpec((1, 64), lambda i: (0, 0)),
            pl.BlockSpec((1, 64), lambda i: (0, 0)),
            pl.BlockSpec((_BS // NPG, _BS), lambda i: (0, 0)),
        ],
        out_specs=[
            pl.BlockSpec((2, _BS, 32), lambda i: (0, i, 0)),
            pl.BlockSpec((_BS // NPG, 64), lambda i: (i, 0)),
        ],
        out_shape=[
            jax.ShapeDtypeStruct((2, N, 32), jnp.float32),
            jax.ShapeDtypeStruct((B * T, 64), jnp.float32),
        ],
    )(t, mom, g, beta, gmat)


def _filter_body(h0, h1, h2, h3, p0, p1, p2, p3,
                 w11, b11, a1, w12, b12,
                 w21, b21, a2, w22, b22, gout):
    hx = jnp.concatenate([h0[...], h1[...], h2[...], h3[...]], axis=1)
    px = jnp.concatenate([p0[...], p1[...], p2[...], p3[...]], axis=1)
    t1 = jnp.dot(hx, w11[...], preferred_element_type=jnp.float32) + b11[...]
    t1 = jnp.where(t1 >= 0, t1, a1[...] * t1)
    z1 = jax.nn.sigmoid(
        jnp.dot(t1, w12[...], preferred_element_type=jnp.float32) + b12[...]) * px
    t2 = jnp.dot(px, w21[...], preferred_element_type=jnp.float32) + b21[...]
    t2 = jnp.where(t2 >= 0, t2, a2[...] * t2)
    z2 = jax.nn.sigmoid(
        jnp.dot(t2, w22[...], preferred_element_type=jnp.float32) + b22[...]) * hx
    gout[...] = jnp.concatenate([z1, z2], axis=1)


def _filter(hp, pp, w11, b11, a1, w12, b12, w21, b21, a2, w22, b22):
    full = lambda shape: pl.BlockSpec(shape, lambda: (0,) * len(shape))
    return pl.pallas_call(
        _filter_body,
        in_specs=[full((B * T, 64))] * 8 + [
            full((256, 256)), full((1, 256)), full((B * T, 1)),
            full((256, 256)), full((1, 256)),
            full((256, 256)), full((1, 256)), full((B * T, 1)),
            full((256, 256)), full((1, 256)),
        ],
        out_specs=full((B * T, 512)),
        out_shape=jax.ShapeDtypeStruct((B * T, 512), jnp.float32),
    )(*hp, *pp, w11, b11, a1, w12, b12, w21, b21, a2, w22, b22)


def _gates1_body(xr, wr, br, out):
    out[...] = jnp.dot(xr[...], wr[...], preferred_element_type=jnp.float32) + br[...]


def _gates1(x, wT, bias):
    return pl.pallas_call(
        _gates1_body,
        grid=(2, 2),
        in_specs=[
            pl.BlockSpec((B * T // 2, 512), lambda i, j: (i, 0)),
            pl.BlockSpec((512, 4 * RNN_H // 2), lambda i, j: (0, j)),
            pl.BlockSpec((1, 4 * RNN_H // 2), lambda i, j: (0, j)),
        ],
        out_specs=pl.BlockSpec((B * T // 2, 4 * RNN_H // 2), lambda i, j: (i, j)),
        out_shape=jax.ShapeDtypeStruct((B * T, 4 * RNN_H), jnp.float32),
    )(x, wT, bias)


def _gates2_body(xa, xb, wa, wb, br, out):
    out[...] = (jnp.dot(xa[...], wa[...], preferred_element_type=jnp.float32)
                + jnp.dot(xb[...], wb[...], preferred_element_type=jnp.float32)
                + br[...])


def _gates2(xa, xb, waT, wbT, bias):
    return pl.pallas_call(
        _gates2_body,
        grid=(2, 2),
        in_specs=[
            pl.BlockSpec((B * T // 2, 512), lambda i, j: (i, 0)),
            pl.BlockSpec((B * T // 2, 512), lambda i, j: (i, 0)),
            pl.BlockSpec((512, 4 * RNN_H // 2), lambda i, j: (0, j)),
            pl.BlockSpec((512, 4 * RNN_H // 2), lambda i, j: (0, j)),
            pl.BlockSpec((1, 4 * RNN_H // 2), lambda i, j: (0, j)),
        ],
        out_specs=pl.BlockSpec((B * T // 2, 4 * RNN_H // 2), lambda i, j: (i, j)),
        out_shape=jax.ShapeDtypeStruct((B * T, 4 * RNN_H), jnp.float32),
    )(xa, xb, waT, wbT, bias)


def _cell(gates, c_prev):
    i_ = jax.nn.sigmoid(gates[:, 0:RNN_H])
    f_ = jax.nn.sigmoid(gates[:, RNN_H:2 * RNN_H])
    g_ = jnp.tanh(gates[:, 2 * RNN_H:3 * RNN_H])
    o_ = jax.nn.sigmoid(gates[:, 3 * RNN_H:4 * RNN_H])
    c_new = f_ * c_prev + i_ * g_
    h_new = o_ * jnp.tanh(c_new)
    return h_new, c_new


def _lstm0_body(gxf, gxb, whf, whb, yf, yb, hf, cf, hb, cb):
    t = pl.program_id(0)

    @pl.when(t == 0)
    def _():
        hf[...] = jnp.zeros_like(hf)
        cf[...] = jnp.zeros_like(cf)
        hb[...] = jnp.zeros_like(hb)
        cb[...] = jnp.zeros_like(cb)

    gf = gxf[0] + jnp.dot(hf[...], whf[...], preferred_element_type=jnp.float32)
    h_new, c_new = _cell(gf, cf[...])
    hf[...] = h_new
    cf[...] = c_new
    yf[0] = h_new

    gb = gxb[0] + jnp.dot(hb[...], whb[...], preferred_element_type=jnp.float32)
    h_new, c_new = _cell(gb, cb[...])
    hb[...] = h_new
    cb[...] = c_new
    yb[0] = h_new


def _lstm0(gxf, gxb, whfT, whbT):
    return pl.pallas_call(
        _lstm0_body,
        grid=(T,),
        in_specs=[
            pl.BlockSpec((1, B, 4 * RNN_H), lambda t: (t, 0, 0)),
            pl.BlockSpec((1, B, 4 * RNN_H), lambda t: (T - 1 - t, 0, 0)),
            pl.BlockSpec((RNN_H, 4 * RNN_H), lambda t: (0, 0)),
            pl.BlockSpec((RNN_H, 4 * RNN_H), lambda t: (0, 0)),
        ],
        out_specs=[
            pl.BlockSpec((1, B, RNN_H), lambda t: (t, 0, 0)),
            pl.BlockSpec((1, B, RNN_H), lambda t: (T - 1 - t, 0, 0)),
        ],
        out_shape=[
            jax.ShapeDtypeStruct((T, B, RNN_H), jnp.float32),
            jax.ShapeDtypeStruct((T, B, RNN_H), jnp.float32),
        ],
        scratch_shapes=[pltpu.VMEM((B, RNN_H), jnp.float32)] * 4,
    )(gxf, gxb, whfT, whbT)


def _lstm1_body(gxf, gxb, whf, whb, hfo, hbo, hf, cf, hb, cb):
    t = pl.program_id(0)

    @pl.when(t == 0)
    def _():
        hf[...] = jnp.zeros_like(hf)
        cf[...] = jnp.zeros_like(cf)
        hb[...] = jnp.zeros_like(hb)
        cb[...] = jnp.zeros_like(cb)

    gf = gxf[0] + jnp.dot(hf[...], whf[...], preferred_element_type=jnp.float32)
    h_new, c_new = _cell(gf, cf[...])
    hf[...] = h_new
    cf[...] = c_new

    gb = gxb[0] + jnp.dot(hb[...], whb[...], preferred_element_type=jnp.float32)
    h_newb, c_newb = _cell(gb, cb[...])
    hb[...] = h_newb
    cb[...] = c_newb

    @pl.when(t == T - 1)
    def _():
        hfo[...] = h_new
        hbo[...] = h_newb


def _lstm1(gxf, gxb, whfT, whbT):
    return pl.pallas_call(
        _lstm1_body,
        grid=(T,),
        in_specs=[
            pl.BlockSpec((1, B, 4 * RNN_H), lambda t: (t, 0, 0)),
            pl.BlockSpec((1, B, 4 * RNN_H), lambda t: (T - 1 - t, 0, 0)),
            pl.BlockSpec((RNN_H, 4 * RNN_H), lambda t: (0, 0)),
            pl.BlockSpec((RNN_H, 4 * RNN_H), lambda t: (0, 0)),
        ],
        out_specs=[
            pl.BlockSpec((B, RNN_H), lambda t: (0, 0)),
            pl.BlockSpec((B, RNN_H), lambda t: (0, 0)),
        ],
        out_shape=[
            jax.ShapeDtypeStruct((B, RNN_H), jnp.float32),
            jax.ShapeDtypeStruct((B, RNN_H), jnp.float32),
        ],
        scratch_shapes=[pltpu.VMEM((B, RNN_H), jnp.float32)] * 4,
    )(gxf, gxb, whfT, whbT)


def _head_body(hb, hf, wfa, wfb, bfc, afc, wcls, bcls, out):
    o = (jnp.dot(hb[...], wfa[...], preferred_element_type=jnp.float32)
         + jnp.dot(hf[...], wfb[...], preferred_element_type=jnp.float32)
         + bfc[...])
    o = jnp.where(o >= 0, o, afc[...] * o)
    out[...] = jnp.dot(o, wcls[...], preferred_element_type=jnp.float32) + bcls[...]


def _head(h1b, h1f, wfaT, wfbT, bfc, afc, wclsT, bcls):
    full = lambda shape: pl.BlockSpec(shape, lambda: (0,) * len(shape))
    return pl.pallas_call(
        _head_body,
        in_specs=[
            full((B, RNN_H)), full((B, RNN_H)),
            full((RNN_H, GCN_OUT)), full((RNN_H, GCN_OUT)),
            full((1, GCN_OUT)), full((1, GCN_OUT)),
            full((GCN_OUT, NUM_CLASSES)), full((1, NUM_CLASSES)),
        ],
        out_specs=full((B, NUM_CLASSES)),
        out_shape=jax.ShapeDtypeStruct((B, NUM_CLASSES), jnp.float32),
    )(h1b, h1f, wfaT, wfbT, bfc, afc, wclsT, bcls)


def _run_gcn_pair(gps, feats, eis, zeros8, ones8, zeros32, gmat):
    xs, degs, srcs, dsts = [], [], [], []
    for gp, feat, ei in zip(gps, feats, eis):
        src = ei[0].astype(jnp.int32)
        dst = ei[1].astype(jnp.int32)
        feat2 = feat.astype(jnp.int32).reshape(NROWS, CH)
        dst2 = dst.reshape(EROWS, CH)
        emb2 = jnp.transpose(gp["emb"].reshape(257, 2, 32), (1, 0, 2)).reshape(514, 32)
        x, degc = _sc_prep(feat2, emb2, dst2, zeros8, ones8)
        xs.append(x)
        degs.append(degc)
        srcs.append(src.reshape(ESROWS, CS))
        dsts.append(dst.reshape(ESROWS, CS))

    pooled = [[], []]
    for i in range(4):
        ss = [_sc_segsum(xs[g].reshape(2 * N, 32), srcs[g], dsts[g], zeros32)
              for g in range(2)]
        for g in range(2):
            gp = gps[g]
            t, mom = _k1(xs[g], ss[g], degs[g],
                         gp["Wself%d" % i].T, gp["Wneigh%d" % i].T,
                         gp["b%d" % i].reshape(1, 64), gp["a%d" % i].reshape(1, 64))
            xs[g], pi = _k2(t, mom, gp["gamma%d" % i].reshape(1, 64),
                            gp["beta%d" % i].reshape(1, 64), gmat)
            pooled[g].append(pi)
    return pooled


def kernel(params, header_feat, header_edge_index, payload_feat,
           payload_edge_index, labels):
    p = params
    zeros8 = jnp.zeros((NSTRIPE, 8), jnp.float32)
    ones8 = jnp.ones((CH, 8), jnp.float32)
    zeros32 = jnp.zeros((NSTRIPE, 32), jnp.float32)
    gmat = jnp.kron(jnp.eye(_BS // NPG, dtype=jnp.float32),
                    jnp.ones((1, NPG), jnp.float32)) * (1.0 / NPG)

    hp, pp = _run_gcn_pair((p["hg"], p["pg"]),
                           (header_feat, payload_feat),
                           (header_edge_index, payload_edge_index),
                           zeros8, ones8, zeros32, gmat)

    a1r = jnp.tile(p["f1a"], B).reshape(B * T, 1)
    a2r = jnp.tile(p["f2a"], B).reshape(B * T, 1)
    g = _filter(hp, pp,
                p["f1W1"].T, p["f1b1"].reshape(1, 256), a1r,
                p["f1W2"].T, p["f1b2"].reshape(1, 256),
                p["f2W1"].T, p["f2b1"].reshape(1, 256), a2r,
                p["f2W2"].T, p["f2b2"].reshape(1, 256))

    xs_flat = jnp.transpose(g.reshape(B, T, 512), (1, 0, 2)).reshape(B * T, 512)

    l0f, l0b = p["lstm0f"], p["lstm0b"]
    gx0f = _gates1(xs_flat, l0f["Wih"].T,
                   (l0f["bih"] + l0f["bhh"]).reshape(1, 4 * RNN_H))
    gx0b = _gates1(xs_flat, l0b["Wih"].T,
                   (l0b["bih"] + l0b["bhh"]).reshape(1, 4 * RNN_H))
    y0f, y0b = _lstm0(gx0f.reshape(T, B, 4 * RNN_H),
                      gx0b.reshape(T, B, 4 * RNN_H),
                      l0f["Whh"].T, l0b["Whh"].T)

    l1f, l1b = p["lstm1f"], p["lstm1b"]
    y0f_flat = y0f.reshape(B * T, RNN_H)
    y0b_flat = y0b.reshape(B * T, RNN_H)
    w1f = l1f["Wih"].T
    w1b = l1b["Wih"].T
    gx1f = _gates2(y0f_flat, y0b_flat, w1f[:RNN_H], w1f[RNN_H:],
                   (l1f["bih"] + l1f["bhh"]).reshape(1, 4 * RNN_H))
    gx1b = _gates2(y0f_flat, y0b_flat, w1b[:RNN_H], w1b[RNN_H:],
                   (l1b["bih"] + l1b["bhh"]).reshape(1, 4 * RNN_H))
    h1f, h1b = _lstm1(gx1f.reshape(T, B, 4 * RNN_H),
                      gx1b.reshape(T, B, 4 * RNN_H),
                      l1f["Whh"].T, l1b["Whh"].T)

    wfcT = p["Wfc"].T
    return _head(h1b, h1f, wfcT[:RNN_H], wfcT[RNN_H:],
                 p["bfc"].reshape(1, GCN_OUT), p["afc"].reshape(1, GCN_OUT),
                 p["Wcls"].T, p["bcls"].reshape(1, NUM_CLASSES))


# speedup vs baseline: 1.0115x; 1.0115x over previous
import functools

import jax
import jax.numpy as jnp
from jax import lax
from jax.experimental import pallas as pl
from jax.experimental.pallas import tpu as pltpu
from jax.experimental.pallas import tpu_sc as plsc

N = 51200
E = 819200
B = 16
T = 50
NPG = 64
H = 64
GCN_OUT = 4 * H
RNN_H = 2 * GCN_OUT
NUM_CLASSES = 10

NS = 16
NSTRIPE = N // NS
CH = 128
KCH = 8
KS = 4
EROWS = E // CH
NROWS = N // CH

_SC_MESH = plsc.VectorSubcoreMesh(core_axis_name="c", subcore_axis_name="s")
_SC_PARAMS = pltpu.CompilerParams(use_tc_tiling_on_sc=False)


KA = 5
KD = 20


def _sc_prep_body(feat2, emb2, dst2, zeros8, ones8,
                  x0_out, deg_out,
                  featv, gidx, rows, dsti, onesv, dacc, gsem, ssem, wsem):
    c = lax.axis_index("c")
    s = lax.axis_index("s")

    rows_per_sub = NSTRIPE // CH

    def block_a(i, carry):
        r0 = s * rows_per_sub + i * KA
        pltpu.sync_copy(feat2.at[pl.ds(r0, KA)], featv)
        for j in range(KA):
            for k in range(CH // 16):
                v = featv[j, pl.ds(k * 16, 16)]
                gidx[j, pl.ds(k * 16, 16)] = v + c * 257
        ghs = [pltpu.async_copy(emb2.at[gidx.at[j]], rows.at[j], gsem)
               for j in range(KA)]
        for h in ghs:
            h.wait()
        whs = [pltpu.async_copy(
                   rows.at[j],
                   x0_out.at[c, pl.ds(s * NSTRIPE + (i * KA + j) * CH, CH)],
                   wsem)
               for j in range(KA)]
        for h in whs:
            h.wait()
        return carry

    lax.fori_loop(0, rows_per_sub // KA, block_a, 0)

    pltpu.sync_copy(zeros8, dacc.at[pl.ds(s * NSTRIPE, NSTRIPE)])
    pltpu.sync_copy(ones8, onesv)
    plsc.subcore_barrier()

    w = s * 2 + c
    rows_per_w = EROWS // 32

    def block_b(i, carry):
        r0 = w * rows_per_w + i * KD
        pltpu.sync_copy(dst2.at[pl.ds(r0, KD)], dsti)
        hs = [pltpu.async_copy(onesv, dacc.at[dsti.at[j]], ssem, add=True)
              for j in range(KD)]
        for h in hs:
            h.wait()
        return carry

    lax.fori_loop(0, rows_per_w // KD, block_b, 0)
    plsc.subcore_barrier()
    pltpu.sync_copy(dacc.at[pl.ds(s * NSTRIPE, NSTRIPE)],
                    deg_out.at[c, pl.ds(s * NSTRIPE, NSTRIPE)])


@functools.partial(jax.jit)
def _sc_prep(feat2, emb2, dst2, zeros8, ones8):
    return pl.kernel(
        _sc_prep_body,
        out_type=(jax.ShapeDtypeStruct((2, N, 32), jnp.float32),
                  jax.ShapeDtypeStruct((2, N, 8), jnp.float32)),
        mesh=_SC_MESH,
        scratch_types=[
            pltpu.VMEM((KA, CH), jnp.int32),
            pltpu.VMEM((KA, CH), jnp.int32),
            pltpu.VMEM((KA, CH, 32), jnp.float32),
            pltpu.VMEM((KD, CH), jnp.int32),
            pltpu.VMEM((CH, 8), jnp.float32),
            pltpu.VMEM_SHARED((N, 8), jnp.float32),
            pltpu.SemaphoreType.DMA,
            pltpu.SemaphoreType.DMA,
            pltpu.SemaphoreType.DMA,
        ],
        compiler_params=_SC_PARAMS,
    )(feat2, emb2, dst2, zeros8, ones8)


CS = 64
KSB = 4
ESROWS = E // CS


def _sc_segsum_body(x2, src2, dst2, zeros32,
                    s_out,
                    srci, dsti, gidx, rows, acc, gsem, ssem):
    c = lax.axis_index("c")
    s = lax.axis_index("s")

    pltpu.sync_copy(zeros32, acc.at[pl.ds(s * NSTRIPE, NSTRIPE)])
    plsc.subcore_barrier()

    cbase = c * N
    rows_per_sub = ESROWS // NS
    nblocks = rows_per_sub // KSB
    base_r = s * rows_per_sub

    def load_idx(b, r0):
        pltpu.sync_copy(src2.at[pl.ds(r0, KSB)], srci.at[b])
        pltpu.sync_copy(dst2.at[pl.ds(r0, KSB)], dsti.at[b])

    def fire_g(b):
        for j in range(KSB):
            for k in range(CS // 16):
                v = srci[b, j, pl.ds(k * 16, 16)]
                gidx[b, j, pl.ds(k * 16, 16)] = v + cbase
        for j in range(KSB):
            pltpu.async_copy(x2.at[gidx.at[b, j]], rows.at[b, j], gsem)

    def wait_g(b):
        for j in range(KSB):
            pltpu.make_async_copy(x2.at[gidx.at[b, j]], rows.at[b, j], gsem).wait()

    def fire_s(b):
        for j in range(KSB):
            pltpu.async_copy(rows.at[b, j], acc.at[dsti.at[b, j]], ssem, add=True)

    def wait_s(b):
        for j in range(KSB):
            pltpu.make_async_copy(rows.at[b, j], acc.at[dsti.at[b, j]], ssem).wait()

    load_idx(0, base_r)
    fire_g(0)

    def blockpair(G, carry):
        for b in range(2):
            load_idx(1 - b, base_r + (2 * G + b + 1) * KSB)
            wait_g(b)
            fire_s(b)
            fire_g(1 - b)
            wait_s(b)
        return carry

    lax.fori_loop(0, nblocks // 2 - 1, blockpair, 0)

    load_idx(1, base_r + (nblocks - 1) * KSB)
    wait_g(0)
    fire_s(0)
    fire_g(1)
    wait_s(0)
    wait_g(1)
    fire_s(1)
    wait_s(1)

    plsc.subcore_barrier()
    pltpu.sync_copy(acc.at[pl.ds(s * NSTRIPE, NSTRIPE)],
                    s_out.at[c, pl.ds(s * NSTRIPE, NSTRIPE)])


@functools.partial(jax.jit)
def _sc_segsum(x2, src2, dst2, zeros32):
    return pl.kernel(
        _sc_segsum_body,
        out_type=jax.ShapeDtypeStruct((2, N, 32), jnp.float32),
        mesh=_SC_MESH,
        scratch_types=[
            pltpu.VMEM((2, KSB, CS), jnp.int32),
            pltpu.VMEM((2, KSB, CS), jnp.int32),
            pltpu.VMEM((2, KSB, CS), jnp.int32),
            pltpu.VMEM((2, KSB, CS, 32), jnp.float32),
            pltpu.VMEM_SHARED((N, 32), jnp.float32),
            pltpu.SemaphoreType.DMA,
            pltpu.SemaphoreType.DMA,
        ],
        compiler_params=_SC_PARAMS,
    )(x2, src2, dst2, zeros32)


_BS = 2048
_GRID1 = N // _BS


def _k1_body(xr, sr, dr, wsr, wnr, br, ar, tout, mout, acc):
    i = pl.program_id(0)
    x64 = jnp.concatenate([xr[0], xr[1]], axis=1)
    s64 = jnp.concatenate([sr[0], sr[1]], axis=1)
    deg = jnp.maximum(dr[0][:, 0:1] + dr[1][:, 0:1], 1.0)
    hn = s64 / deg
    t = (jnp.dot(x64, wsr[...], preferred_element_type=jnp.float32)
         + jnp.dot(hn, wnr[...], preferred_element_type=jnp.float32)
         + br[...])
    t = jnp.where(t >= 0, t, ar[...] * t)
    tout[...] = t

    @pl.when(i == 0)
    def _():
        acc[...] = jnp.zeros_like(acc)

    acc[0:1, :] += jnp.sum(t, axis=0, keepdims=True)
    acc[1:2, :] += jnp.sum(t * t, axis=0, keepdims=True)

    @pl.when(i == _GRID1 - 1)
    def _():
        mout[...] = acc[...]


def _k1(x, s, degc, wsT, wnT, b, a):
    return pl.pallas_call(
        _k1_body,
        grid=(_GRID1,),
        in_specs=[
            pl.BlockSpec((2, _BS, 32), lambda i: (0, i, 0)),
            pl.BlockSpec((2, _BS, 32), lambda i: (0, i, 0)),
            pl.BlockSpec((2, _BS, 8), lambda i: (0, i, 0)),
            pl.BlockSpec((64, 64), lambda i: (0, 0)),
            pl.BlockSpec((64, 64), lambda i: (0, 0)),
            pl.BlockSpec((1, 64), lambda i: (0, 0)),
            pl.BlockSpec((1, 64), lambda i: (0, 0)),
        ],
        out_specs=[
            pl.BlockSpec((_BS, 64), lambda i: (i, 0)),
            pl.BlockSpec((2, 64), lambda i: (0, 0)),
        ],
        out_shape=[
            jax.ShapeDtypeStruct((N, 64), jnp.float32),
            jax.ShapeDtypeStruct((2, 64), jnp.float32),
        ],
        scratch_shapes=[pltpu.VMEM((2, 64), jnp.float32)],
    )(x, s, degc, wsT, wnT, b, a)


def _k2_body(tr, momr, gr, betr, gmr, xout, pout):
    m = momr[0:1, :] * (1.0 / N)
    v = momr[1:2, :] * (1.0 / N) - m * m
    sc = gr[...] * lax.rsqrt(v + 1e-5)
    xn = (tr[...] - m) * sc + betr[...]
    xout[...] = jnp.stack([xn[:, :32], xn[:, 32:]], axis=0)
    pout[...] = jnp.dot(gmr[...], xn, preferred_element_type=jnp.float32)


def _k2(t, mom, g, beta, gmat):
    return pl.pallas_call(
        _k2_body,
        grid=(_GRID1,),
        in_specs=[
            pl.BlockSpec((_BS, 64), lambda i: (i, 0)),
            pl.BlockSpec((2, 64), lambda i: (0, 0)),
            pl.BlockSpec((1, 64), lambda i: (0, 0)),
            pl.BlockSpec((1, 64), lambda i: (0, 0)),
            pl.BlockSpec((_BS // NPG, _BS), lambda i: (0, 0)),
        ],
        out_specs=[
            pl.BlockSpec((2, _BS, 32), lambda i: (0, i, 0)),
            pl.BlockSpec((_BS // NPG, 64), lambda i: (i, 0)),
        ],
        out_shape=[
            jax.ShapeDtypeStruct((2, N, 32), jnp.float32),
            jax.ShapeDtypeStruct((B * T, 64), jnp.float32),
        ],
    )(t, mom, g, beta, gmat)


def _filter_body(h0, h1, h2, h3, p0, p1, p2, p3,
                 w11, b11, a1, w12, b12,
                 w21, b21, a2, w22, b22, gout):
    hx = jnp.concatenate([h0[...], h1[...], h2[...], h3[...]], axis=1)
    px = jnp.concatenate([p0[...], p1[...], p2[...], p3[...]], axis=1)
    t1 = jnp.dot(hx, w11[...], preferred_element_type=jnp.float32) + b11[...]
    t1 = jnp.where(t1 >= 0, t1, a1[...] * t1)
    z1 = jax.nn.sigmoid(
        jnp.dot(t1, w12[...], preferred_element_type=jnp.float32) + b12[...]) * px
    t2 = jnp.dot(px, w21[...], preferred_element_type=jnp.float32) + b21[...]
    t2 = jnp.where(t2 >= 0, t2, a2[...] * t2)
    z2 = jax.nn.sigmoid(
        jnp.dot(t2, w22[...], preferred_element_type=jnp.float32) + b22[...]) * hx
    gout[...] = jnp.concatenate([z1, z2], axis=1)


def _filter(hp, pp, w11, b11, a1, w12, b12, w21, b21, a2, w22, b22):
    full = lambda shape: pl.BlockSpec(shape, lambda: (0,) * len(shape))
    return pl.pallas_call(
        _filter_body,
        in_specs=[full((B * T, 64))] * 8 + [
            full((256, 256)), full((1, 256)), full((B * T, 1)),
            full((256, 256)), full((1, 256)),
            full((256, 256)), full((1, 256)), full((B * T, 1)),
            full((256, 256)), full((1, 256)),
        ],
        out_specs=full((B * T, 512)),
        out_shape=jax.ShapeDtypeStruct((B * T, 512), jnp.float32),
    )(*hp, *pp, w11, b11, a1, w12, b12, w21, b21, a2, w22, b22)


def _gates1_body(xr, wr, br, out):
    out[...] = jnp.dot(xr[...], wr[...], preferred_element_type=jnp.float32) + br[...]


def _gates1(x, wT, bias):
    return pl.pallas_call(
        _gates1_body,
        grid=(2, 2),
        in_specs=[
            pl.BlockSpec((B * T // 2, 512), lambda i, j: (i, 0)),
            pl.BlockSpec((512, 4 * RNN_H // 2), lambda i, j: (0, j)),
            pl.BlockSpec((1, 4 * RNN_H // 2), lambda i, j: (0, j)),
        ],
        out_specs=pl.BlockSpec((B * T // 2, 4 * RNN_H // 2), lambda i, j: (i, j)),
        out_shape=jax.ShapeDtypeStruct((B * T, 4 * RNN_H), jnp.float32),
    )(x, wT, bias)


def _gates2_body(xa, xb, wa, wb, br, out):
    out[...] = (jnp.dot(xa[...], wa[...], preferred_element_type=jnp.float32)
                + jnp.dot(xb[...], wb[...], preferred_element_type=jnp.float32)
                + br[...])


def _gates2(xa, xb, waT, wbT, bias):
    return pl.pallas_call(
        _gates2_body,
        grid=(2, 2),
        in_specs=[
            pl.BlockSpec((B * T // 2, 512), lambda i, j: (i, 0)),
            pl.BlockSpec((B * T // 2, 512), lambda i, j: (i, 0)),
            pl.BlockSpec((512, 4 * RNN_H // 2), lambda i, j: (0, j)),
            pl.BlockSpec((512, 4 * RNN_H // 2), lambda i, j: (0, j)),
            pl.BlockSpec((1, 4 * RNN_H // 2), lambda i, j: (0, j)),
        ],
        out_specs=pl.BlockSpec((B * T // 2, 4 * RNN_H // 2), lambda i, j: (i, j)),
        out_shape=jax.ShapeDtypeStruct((B * T, 4 * RNN_H), jnp.float32),
    )(xa, xb, waT, wbT, bias)


def _cell(gates, c_prev):
    i_ = jax.nn.sigmoid(gates[:, 0:RNN_H])
    f_ = jax.nn.sigmoid(gates[:, RNN_H:2 * RNN_H])
    g_ = jnp.tanh(gates[:, 2 * RNN_H:3 * RNN_H])
    o_ = jax.nn.sigmoid(gates[:, 3 * RNN_H:4 * RNN_H])
    c_new = f_ * c_prev + i_ * g_
    h_new = o_ * jnp.tanh(c_new)
    return h_new, c_new


def _lstm0_body(gxf, gxb, whf, whb, yf, yb, hf, cf, hb, cb):
    t = pl.program_id(0)

    @pl.when(t == 0)
    def _():
        hf[...] = jnp.zeros_like(hf)
        cf[...] = jnp.zeros_like(cf)
        hb[...] = jnp.zeros_like(hb)
        cb[...] = jnp.zeros_like(cb)

    gf = gxf[0] + jnp.dot(hf[...], whf[...], preferred_element_type=jnp.float32)
    h_new, c_new = _cell(gf, cf[...])
    hf[...] = h_new
    cf[...] = c_new
    yf[0] = h_new

    gb = gxb[0] + jnp.dot(hb[...], whb[...], preferred_element_type=jnp.float32)
    h_new, c_new = _cell(gb, cb[...])
    hb[...] = h_new
    cb[...] = c_new
    yb[0] = h_new


def _lstm0(gxf, gxb, whfT, whbT):
    return pl.pallas_call(
        _lstm0_body,
        grid=(T,),
        in_specs=[
            pl.BlockSpec((1, B, 4 * RNN_H), lambda t: (t, 0, 0)),
            pl.BlockSpec((1, B, 4 * RNN_H), lambda t: (T - 1 - t, 0, 0)),
            pl.BlockSpec((RNN_H, 4 * RNN_H), lambda t: (0, 0)),
            pl.BlockSpec((RNN_H, 4 * RNN_H), lambda t: (0, 0)),
        ],
        out_specs=[
            pl.BlockSpec((1, B, RNN_H), lambda t: (t, 0, 0)),
            pl.BlockSpec((1, B, RNN_H), lambda t: (T - 1 - t, 0, 0)),
        ],
        out_shape=[
            jax.ShapeDtypeStruct((T, B, RNN_H), jnp.float32),
            jax.ShapeDtypeStruct((T, B, RNN_H), jnp.float32),
        ],
        scratch_shapes=[pltpu.VMEM((B, RNN_H), jnp.float32)] * 4,
    )(gxf, gxb, whfT, whbT)


def _lstm1_body(gxf, gxb, whf, whb, hfo, hbo, hf, cf, hb, cb):
    t = pl.program_id(0)

    @pl.when(t == 0)
    def _():
        hf[...] = jnp.zeros_like(hf)
        cf[...] = jnp.zeros_like(cf)
        hb[...] = jnp.zeros_like(hb)
        cb[...] = jnp.zeros_like(cb)

    gf = gxf[0] + jnp.dot(hf[...], whf[...], preferred_element_type=jnp.float32)
    h_new, c_new = _cell(gf, cf[...])
    hf[...] = h_new
    cf[...] = c_new

    gb = gxb[0] + jnp.dot(hb[...], whb[...], preferred_element_type=jnp.float32)
    h_newb, c_newb = _cell(gb, cb[...])
    hb[...] = h_newb
    cb[...] = c_newb

    @pl.when(t == T - 1)
    def _():
        hfo[...] = h_new
        hbo[...] = h_newb


def _lstm1(gxf, gxb, whfT, whbT):
    return pl.pallas_call(
        _lstm1_body,
        grid=(T,),
        in_specs=[
            pl.BlockSpec((1, B, 4 * RNN_H), lambda t: (t, 0, 0)),
            pl.BlockSpec((1, B, 4 * RNN_H), lambda t: (T - 1 - t, 0, 0)),
            pl.BlockSpec((RNN_H, 4 * RNN_H), lambda t: (0, 0)),
            pl.BlockSpec((RNN_H, 4 * RNN_H), lambda t: (0, 0)),
        ],
        out_specs=[
            pl.BlockSpec((B, RNN_H), lambda t: (0, 0)),
            pl.BlockSpec((B, RNN_H), lambda t: (0, 0)),
        ],
        out_shape=[
            jax.ShapeDtypeStruct((B, RNN_H), jnp.float32),
            jax.ShapeDtypeStruct((B, RNN_H), jnp.float32),
        ],
        scratch_shapes=[pltpu.VMEM((B, RNN_H), jnp.float32)] * 4,
    )(gxf, gxb, whfT, whbT)


def _head_body(hb, hf, wfa, wfb, bfc, afc, wcls, bcls, out):
    o = (jnp.dot(hb[...], wfa[...], preferred_element_type=jnp.float32)
         + jnp.dot(hf[...], wfb[...], preferred_element_type=jnp.float32)
         + bfc[...])
    o = jnp.where(o >= 0, o, afc[...] * o)
    out[...] = jnp.dot(o, wcls[...], preferred_element_type=jnp.float32) + bcls[...]


def _head(h1b, h1f, wfaT, wfbT, bfc, afc, wclsT, bcls):
    full = lambda shape: pl.BlockSpec(shape, lambda: (0,) * len(shape))
    return pl.pallas_call(
        _head_body,
        in_specs=[
            full((B, RNN_H)), full((B, RNN_H)),
            full((RNN_H, GCN_OUT)), full((RNN_H, GCN_OUT)),
            full((1, GCN_OUT)), full((1, GCN_OUT)),
            full((GCN_OUT, NUM_CLASSES)), full((1, NUM_CLASSES)),
        ],
        out_specs=full((B, NUM_CLASSES)),
        out_shape=jax.ShapeDtypeStruct((B, NUM_CLASSES), jnp.float32),
    )(h1b, h1f, wfaT, wfbT, bfc, afc, wclsT, bcls)


def _run_gcn_pair(gps, feats, eis, zeros8, ones8, zeros32, gmat):
    xs, degs, srcs, dsts = [], [], [], []
    for gp, feat, ei in zip(gps, feats, eis):
        src = ei[0].astype(jnp.int32)
        dst = ei[1].astype(jnp.int32)
        feat2 = feat.astype(jnp.int32).reshape(NROWS, CH)
        dst2 = dst.reshape(EROWS, CH)
        emb2 = jnp.transpose(gp["emb"].reshape(257, 2, 32), (1, 0, 2)).reshape(514, 32)
        x, degc = _sc_prep(feat2, emb2, dst2, zeros8, ones8)
        xs.append(x)
        degs.append(degc)
        srcs.append(src.reshape(ESROWS, CS))
        dsts.append(dst.reshape(ESROWS, CS))

    pooled = [[], []]
    for i in range(4):
        ss = [_sc_segsum(xs[g].reshape(2 * N, 32), srcs[g], dsts[g], zeros32)
              for g in range(2)]
        for g in range(2):
            gp = gps[g]
            t, mom = _k1(xs[g], ss[g], degs[g],
                         gp["Wself%d" % i].T, gp["Wneigh%d" % i].T,
                         gp["b%d" % i].reshape(1, 64), gp["a%d" % i].reshape(1, 64))
            xs[g], pi = _k2(t, mom, gp["gamma%d" % i].reshape(1, 64),
                            gp["beta%d" % i].reshape(1, 64), gmat)
            pooled[g].append(pi)
    return pooled


def kernel(params, header_feat, header_edge_index, payload_feat,
           payload_edge_index, labels):
    p = params
    zeros8 = jnp.zeros((NSTRIPE, 8), jnp.float32)
    ones8 = jnp.ones((CH, 8), jnp.float32)
    zeros32 = jnp.zeros((NSTRIPE, 32), jnp.float32)
    gmat = jnp.kron(jnp.eye(_BS // NPG, dtype=jnp.float32),
                    jnp.ones((1, NPG), jnp.float32)) * (1.0 / NPG)

    hp, pp = _run_gcn_pair((p["hg"], p["pg"]),
                           (header_feat, payload_feat),
                           (header_edge_index, payload_edge_index),
                           zeros8, ones8, zeros32, gmat)

    a1r = jnp.tile(p["f1a"], B).reshape(B * T, 1)
    a2r = jnp.tile(p["f2a"], B).reshape(B * T, 1)
    g = _filter(hp, pp,
                p["f1W1"].T, p["f1b1"].reshape(1, 256), a1r,
                p["f1W2"].T, p["f1b2"].reshape(1, 256),
                p["f2W1"].T, p["f2b1"].reshape(1, 256), a2r,
                p["f2W2"].T, p["f2b2"].reshape(1, 256))

    xs_flat = jnp.transpose(g.reshape(B, T, 512), (1, 0, 2)).reshape(B * T, 512)

    l0f, l0b = p["lstm0f"], p["lstm0b"]
    gx0f = _gates1(xs_flat, l0f["Wih"].T,
                   (l0f["bih"] + l0f["bhh"]).reshape(1, 4 * RNN_H))
    gx0b = _gates1(xs_flat, l0b["Wih"].T,
                   (l0b["bih"] + l0b["bhh"]).reshape(1, 4 * RNN_H))
    y0f, y0b = _lstm0(gx0f.reshape(T, B, 4 * RNN_H),
                      gx0b.reshape(T, B, 4 * RNN_H),
                      l0f["Whh"].T, l0b["Whh"].T)

    l1f, l1b = p["lstm1f"], p["lstm1b"]
    y0f_flat = y0f.reshape(B * T, RNN_H)
    y0b_flat = y0b.reshape(B * T, RNN_H)
    w1f = l1f["Wih"].T
    w1b = l1b["Wih"].T
    gx1f = _gates2(y0f_flat, y0b_flat, w1f[:RNN_H], w1f[RNN_H:],
                   (l1f["bih"] + l1f["bhh"]).reshape(1, 4 * RNN_H))
    gx1b = _gates2(y0f_flat, y0b_flat, w1b[:RNN_H], w1b[RNN_H:],
                   (l1b["bih"] + l1b["bhh"]).reshape(1, 4 * RNN_H))
    h1f, h1b = _lstm1(gx1f.reshape(T, B, 4 * RNN_H),
                      gx1b.reshape(T, B, 4 * RNN_H),
                      l1f["Whh"].T, l1b["Whh"].T)

    wfcT = p["Wfc"].T
    return _head(h1b, h1f, wfcT[:RNN_H], wfcT[RNN_H:],
                 p["bfc"].reshape(1, GCN_OUT), p["afc"].reshape(1, GCN_OUT),
                 p["Wcls"].T, p["bcls"].reshape(1, NUM_CLASSES))


# speedup vs baseline: 1.1173x; 1.1046x over previous
import functools

import jax
import jax.numpy as jnp
from jax import lax
from jax.experimental import pallas as pl
from jax.experimental.pallas import tpu as pltpu
from jax.experimental.pallas import tpu_sc as plsc

N = 51200
E = 819200
B = 16
T = 50
NPG = 64
H = 64
GCN_OUT = 4 * H
RNN_H = 2 * GCN_OUT
NUM_CLASSES = 10

NS = 16
NSTRIPE = N // NS
CH = 128
KCH = 8
KS = 4
EROWS = E // CH
NROWS = N // CH

_SC_MESH = plsc.VectorSubcoreMesh(core_axis_name="c", subcore_axis_name="s")
_SC_PARAMS = pltpu.CompilerParams(use_tc_tiling_on_sc=False)


KA = 5
KD = 20


def _sc_prep_body(feat2, emb2, dst2, zeros8, ones8,
                  x0_out, deg_out,
                  featv, gidx, rows, dsti, onesv, dacc, gsem, ssem, wsem):
    c = lax.axis_index("c")
    s = lax.axis_index("s")

    rows_per_sub = NSTRIPE // CH

    def block_a(i, carry):
        r0 = s * rows_per_sub + i * KA
        pltpu.sync_copy(feat2.at[pl.ds(r0, KA)], featv)
        for j in range(KA):
            for k in range(CH // 16):
                v = featv[j, pl.ds(k * 16, 16)]
                gidx[j, pl.ds(k * 16, 16)] = v + c * 257
        ghs = [pltpu.async_copy(emb2.at[gidx.at[j]], rows.at[j], gsem)
               for j in range(KA)]
        for h in ghs:
            h.wait()
        whs = [pltpu.async_copy(
                   rows.at[j],
                   x0_out.at[c, pl.ds(s * NSTRIPE + (i * KA + j) * CH, CH)],
                   wsem)
               for j in range(KA)]
        for h in whs:
            h.wait()
        return carry

    lax.fori_loop(0, rows_per_sub // KA, block_a, 0)

    pltpu.sync_copy(zeros8, dacc.at[pl.ds(s * NSTRIPE, NSTRIPE)])
    pltpu.sync_copy(ones8, onesv)
    plsc.subcore_barrier()

    w = s * 2 + c
    rows_per_w = EROWS // 32

    def block_b(i, carry):
        r0 = w * rows_per_w + i * KD
        pltpu.sync_copy(dst2.at[pl.ds(r0, KD)], dsti)
        hs = [pltpu.async_copy(onesv, dacc.at[dsti.at[j]], ssem, add=True)
              for j in range(KD)]
        for h in hs:
            h.wait()
        return carry

    lax.fori_loop(0, rows_per_w // KD, block_b, 0)
    plsc.subcore_barrier()
    pltpu.sync_copy(dacc.at[pl.ds(s * NSTRIPE, NSTRIPE)],
                    deg_out.at[c, pl.ds(s * NSTRIPE, NSTRIPE)])


@functools.partial(jax.jit)
def _sc_prep(feat2, emb2, dst2, zeros8, ones8):
    return pl.kernel(
        _sc_prep_body,
        out_type=(jax.ShapeDtypeStruct((2, N, 32), jnp.float32),
                  jax.ShapeDtypeStruct((2, N, 8), jnp.float32)),
        mesh=_SC_MESH,
        scratch_types=[
            pltpu.VMEM((KA, CH), jnp.int32),
            pltpu.VMEM((KA, CH), jnp.int32),
            pltpu.VMEM((KA, CH, 32), jnp.float32),
            pltpu.VMEM((KD, CH), jnp.int32),
            pltpu.VMEM((CH, 8), jnp.float32),
            pltpu.VMEM_SHARED((N, 8), jnp.float32),
            pltpu.SemaphoreType.DMA,
            pltpu.SemaphoreType.DMA,
            pltpu.SemaphoreType.DMA,
        ],
        compiler_params=_SC_PARAMS,
    )(feat2, emb2, dst2, zeros8, ones8)


CS = 80
KSB = 4
ESROWS = E // CS


def _sc_segsum_body(x2, src2, dst2, zeros32,
                    s_out,
                    srci, dsti, gidx, rows, acc, gsem, ssem):
    c = lax.axis_index("c")
    s = lax.axis_index("s")

    pltpu.sync_copy(zeros32, acc.at[pl.ds(s * NSTRIPE, NSTRIPE)])
    plsc.subcore_barrier()

    cbase = c * N
    rows_per_sub = ESROWS // NS
    nblocks = rows_per_sub // KSB
    base_r = s * rows_per_sub

    def load_idx(b, r0):
        pltpu.sync_copy(src2.at[pl.ds(r0, KSB)], srci.at[b])
        pltpu.sync_copy(dst2.at[pl.ds(r0, KSB)], dsti.at[b])

    def fire_g(b):
        for j in range(KSB):
            for k in range(CS // 16):
                v = srci[b, j, pl.ds(k * 16, 16)]
                gidx[b, j, pl.ds(k * 16, 16)] = v + cbase
        for j in range(KSB):
            pltpu.async_copy(x2.at[gidx.at[b, j]], rows.at[b, j], gsem)

    def wait_g(b):
        for j in range(KSB):
            pltpu.make_async_copy(x2.at[gidx.at[b, j]], rows.at[b, j], gsem).wait()

    def fire_s(b):
        for j in range(KSB):
            pltpu.async_copy(rows.at[b, j], acc.at[dsti.at[b, j]], ssem, add=True)

    def wait_s(b):
        for j in range(KSB):
            pltpu.make_async_copy(rows.at[b, j], acc.at[dsti.at[b, j]], ssem).wait()

    load_idx(0, base_r)
    fire_g(0)

    def blockpair(G, carry):
        for b in range(2):
            load_idx(1 - b, base_r + (2 * G + b + 1) * KSB)
            wait_g(b)
            fire_s(b)
            fire_g(1 - b)
            wait_s(b)
        return carry

    lax.fori_loop(0, nblocks // 2 - 1, blockpair, 0)

    load_idx(1, base_r + (nblocks - 1) * KSB)
    wait_g(0)
    fire_s(0)
    fire_g(1)
    wait_s(0)
    wait_g(1)
    fire_s(1)
    wait_s(1)

    plsc.subcore_barrier()
    pltpu.sync_copy(acc.at[pl.ds(s * NSTRIPE, NSTRIPE)],
                    s_out.at[c, pl.ds(s * NSTRIPE, NSTRIPE)])


@functools.partial(jax.jit)
def _sc_segsum(x2, src2, dst2, zeros32):
    return pl.kernel(
        _sc_segsum_body,
        out_type=jax.ShapeDtypeStruct((2, N, 32), jnp.float32),
        mesh=_SC_MESH,
        scratch_types=[
            pltpu.VMEM((2, KSB, CS), jnp.int32),
            pltpu.VMEM((2, KSB, CS), jnp.int32),
            pltpu.VMEM((2, KSB, CS), jnp.int32),
            pltpu.VMEM((2, KSB, CS, 32), jnp.float32),
            pltpu.VMEM_SHARED((N, 32), jnp.float32),
            pltpu.SemaphoreType.DMA,
            pltpu.SemaphoreType.DMA,
        ],
        compiler_params=_SC_PARAMS,
    )(x2, src2, dst2, zeros32)


_BS = 2048
_GRID1 = N // _BS


def _k1_body(xr, sr, dr, wsr, wnr, br, ar, tout, mout, acc):
    i = pl.program_id(0)
    x64 = jnp.concatenate([xr[0], xr[1]], axis=1)
    s64 = jnp.concatenate([sr[0], sr[1]], axis=1)
    deg = jnp.maximum(dr[0][:, 0:1] + dr[1][:, 0:1], 1.0)
    hn = s64 / deg
    t = (jnp.dot(x64, wsr[...], preferred_element_type=jnp.float32)
         + jnp.dot(hn, wnr[...], preferred_element_type=jnp.float32)
         + br[...])
    t = jnp.where(t >= 0, t, ar[...] * t)
    tout[...] = t

    @pl.when(i == 0)
    def _():
        acc[...] = jnp.zeros_like(acc)

    acc[0:1, :] += jnp.sum(t, axis=0, keepdims=True)
    acc[1:2, :] += jnp.sum(t * t, axis=0, keepdims=True)

    @pl.when(i == _GRID1 - 1)
    def _():
        mout[...] = acc[...]


def _k1(x, s, degc, wsT, wnT, b, a):
    return pl.pallas_call(
        _k1_body,
        grid=(_GRID1,),
        in_specs=[
            pl.BlockSpec((2, _BS, 32), lambda i: (0, i, 0)),
            pl.BlockSpec((2, _BS, 32), lambda i: (0, i, 0)),
            pl.BlockSpec((2, _BS, 8), lambda i: (0, i, 0)),
            pl.BlockSpec((64, 64), lambda i: (0, 0)),
            pl.BlockSpec((64, 64), lambda i: (0, 0)),
            pl.BlockSpec((1, 64), lambda i: (0, 0)),
            pl.BlockSpec((1, 64), lambda i: (0, 0)),
        ],
        out_specs=[
            pl.BlockSpec((_BS, 64), lambda i: (i, 0)),
            pl.BlockSpec((2, 64), lambda i: (0, 0)),
        ],
        out_shape=[
            jax.ShapeDtypeStruct((N, 64), jnp.float32),
            jax.ShapeDtypeStruct((2, 64), jnp.float32),
        ],
        scratch_shapes=[pltpu.VMEM((2, 64), jnp.float32)],
    )(x, s, degc, wsT, wnT, b, a)


def _k2_body(tr, momr, gr, betr, gmr, xout, pout):
    m = momr[0:1, :] * (1.0 / N)
    v = momr[1:2, :] * (1.0 / N) - m * m
    sc = gr[...] * lax.rsqrt(v + 1e-5)
    xn = (tr[...] - m) * sc + betr[...]
    xout[...] = jnp.stack([xn[:, :32], xn[:, 32:]], axis=0)
    pout[...] = jnp.dot(gmr[...], xn, preferred_element_type=jnp.float32)


def _k2(t, mom, g, beta, gmat):
    return pl.pallas_call(
        _k2_body,
        grid=(_GRID1,),
        in_specs=[
            pl.BlockSpec((_BS, 64), lambda i: (i, 0)),
            pl.BlockSpec((2, 64), lambda i: (0, 0)),
            pl.BlockSpec((1, 64), lambda i: (0, 0)),
            pl.BlockSpec((1, 64), lambda i: (0, 0)),
            pl.BlockSpec((_BS // NPG, _BS), lambda i: (0, 0)),
        ],
        out_specs=[
            pl.BlockSpec((2, _BS, 32), lambda i: (0, i, 0)),
            pl.BlockSpec((_BS // NPG, 64), lambda i: (i, 0)),
        ],
        out_shape=[
            jax.ShapeDtypeStruct((2, N, 32), jnp.float32),
            jax.ShapeDtypeStruct((B * T, 64), jnp.float32),
        ],
    )(t, mom, g, beta, gmat)


def _filter_body(h0, h1, h2, h3, p0, p1, p2, p3,
                 w11, b11, a1, w12, b12,
                 w21, b21, a2, w22, b22, gout):
    hx = jnp.concatenate([h0[...], h1[...], h2[...], h3[...]], axis=1)
    px = jnp.concatenate([p0[...], p1[...], p2[...], p3[...]], axis=1)
    t1 = jnp.dot(hx, w11[...], preferred_element_type=jnp.float32) + b11[...]
    t1 = jnp.where(t1 >= 0, t1, a1[...] * t1)
    z1 = jax.nn.sigmoid(
        jnp.dot(t1, w12[...], preferred_element_type=jnp.float32) + b12[...]) * px
    t2 = jnp.dot(px, w21[...], preferred_element_type=jnp.float32) + b21[...]
    t2 = jnp.where(t2 >= 0, t2, a2[...] * t2)
    z2 = jax.nn.sigmoid(
        jnp.dot(t2, w22[...], preferred_element_type=jnp.float32) + b22[...]) * hx
    gout[...] = jnp.concatenate([z1, z2], axis=1)


def _filter(hp, pp, w11, b11, a1, w12, b12, w21, b21, a2, w22, b22):
    full = lambda shape: pl.BlockSpec(shape, lambda: (0,) * len(shape))
    return pl.pallas_call(
        _filter_body,
        in_specs=[full((B * T, 64))] * 8 + [
            full((256, 256)), full((1, 256)), full((B * T, 1)),
            full((256, 256)), full((1, 256)),
            full((256, 256)), full((1, 256)), full((B * T, 1)),
            full((256, 256)), full((1, 256)),
        ],
        out_specs=full((B * T, 512)),
        out_shape=jax.ShapeDtypeStruct((B * T, 512), jnp.float32),
    )(*hp, *pp, w11, b11, a1, w12, b12, w21, b21, a2, w22, b22)


def _gates1_body(xr, wr, br, out):
    out[...] = jnp.dot(xr[...], wr[...], preferred_element_type=jnp.float32) + br[...]


def _gates1(x, wT, bias):
    return pl.pallas_call(
        _gates1_body,
        grid=(2, 2),
        in_specs=[
            pl.BlockSpec((B * T // 2, 512), lambda i, j: (i, 0)),
            pl.BlockSpec((512, 4 * RNN_H // 2), lambda i, j: (0, j)),
            pl.BlockSpec((1, 4 * RNN_H // 2), lambda i, j: (0, j)),
        ],
        out_specs=pl.BlockSpec((B * T // 2, 4 * RNN_H // 2), lambda i, j: (i, j)),
        out_shape=jax.ShapeDtypeStruct((B * T, 4 * RNN_H), jnp.float32),
    )(x, wT, bias)


def _gates2_body(xa, xb, wa, wb, br, out):
    out[...] = (jnp.dot(xa[...], wa[...], preferred_element_type=jnp.float32)
                + jnp.dot(xb[...], wb[...], preferred_element_type=jnp.float32)
                + br[...])


def _gates2(xa, xb, waT, wbT, bias):
    return pl.pallas_call(
        _gates2_body,
        grid=(2, 2),
        in_specs=[
            pl.BlockSpec((B * T // 2, 512), lambda i, j: (i, 0)),
            pl.BlockSpec((B * T // 2, 512), lambda i, j: (i, 0)),
            pl.BlockSpec((512, 4 * RNN_H // 2), lambda i, j: (0, j)),
            pl.BlockSpec((512, 4 * RNN_H // 2), lambda i, j: (0, j)),
            pl.BlockSpec((1, 4 * RNN_H // 2), lambda i, j: (0, j)),
        ],
        out_specs=pl.BlockSpec((B * T // 2, 4 * RNN_H // 2), lambda i, j: (i, j)),
        out_shape=jax.ShapeDtypeStruct((B * T, 4 * RNN_H), jnp.float32),
    )(xa, xb, waT, wbT, bias)


def _cell(gates, c_prev):
    i_ = jax.nn.sigmoid(gates[:, 0:RNN_H])
    f_ = jax.nn.sigmoid(gates[:, RNN_H:2 * RNN_H])
    g_ = jnp.tanh(gates[:, 2 * RNN_H:3 * RNN_H])
    o_ = jax.nn.sigmoid(gates[:, 3 * RNN_H:4 * RNN_H])
    c_new = f_ * c_prev + i_ * g_
    h_new = o_ * jnp.tanh(c_new)
    return h_new, c_new


def _lstm0_body(gxf, gxb, whf, whb, yf, yb, hf, cf, hb, cb):
    t = pl.program_id(0)

    @pl.when(t == 0)
    def _():
        hf[...] = jnp.zeros_like(hf)
        cf[...] = jnp.zeros_like(cf)
        hb[...] = jnp.zeros_like(hb)
        cb[...] = jnp.zeros_like(cb)

    gf = gxf[0] + jnp.dot(hf[...], whf[...], preferred_element_type=jnp.float32)
    h_new, c_new = _cell(gf, cf[...])
    hf[...] = h_new
    cf[...] = c_new
    yf[0] = h_new

    gb = gxb[0] + jnp.dot(hb[...], whb[...], preferred_element_type=jnp.float32)
    h_new, c_new = _cell(gb, cb[...])
    hb[...] = h_new
    cb[...] = c_new
    yb[0] = h_new


def _lstm0(gxf, gxb, whfT, whbT):
    return pl.pallas_call(
        _lstm0_body,
        grid=(T,),
        in_specs=[
            pl.BlockSpec((1, B, 4 * RNN_H), lambda t: (t, 0, 0)),
            pl.BlockSpec((1, B, 4 * RNN_H), lambda t: (T - 1 - t, 0, 0)),
            pl.BlockSpec((RNN_H, 4 * RNN_H), lambda t: (0, 0)),
            pl.BlockSpec((RNN_H, 4 * RNN_H), lambda t: (0, 0)),
        ],
        out_specs=[
            pl.BlockSpec((1, B, RNN_H), lambda t: (t, 0, 0)),
            pl.BlockSpec((1, B, RNN_H), lambda t: (T - 1 - t, 0, 0)),
        ],
        out_shape=[
            jax.ShapeDtypeStruct((T, B, RNN_H), jnp.float32),
            jax.ShapeDtypeStruct((T, B, RNN_H), jnp.float32),
        ],
        scratch_shapes=[pltpu.VMEM((B, RNN_H), jnp.float32)] * 4,
    )(gxf, gxb, whfT, whbT)


def _lstm1_body(gxf, gxb, whf, whb, hfo, hbo, hf, cf, hb, cb):
    t = pl.program_id(0)

    @pl.when(t == 0)
    def _():
        hf[...] = jnp.zeros_like(hf)
        cf[...] = jnp.zeros_like(cf)
        hb[...] = jnp.zeros_like(hb)
        cb[...] = jnp.zeros_like(cb)

    gf = gxf[0] + jnp.dot(hf[...], whf[...], preferred_element_type=jnp.float32)
    h_new, c_new = _cell(gf, cf[...])
    hf[...] = h_new
    cf[...] = c_new

    gb = gxb[0] + jnp.dot(hb[...], whb[...], preferred_element_type=jnp.float32)
    h_newb, c_newb = _cell(gb, cb[...])
    hb[...] = h_newb
    cb[...] = c_newb

    @pl.when(t == T - 1)
    def _():
        hfo[...] = h_new
        hbo[...] = h_newb


def _lstm1(gxf, gxb, whfT, whbT):
    return pl.pallas_call(
        _lstm1_body,
        grid=(T,),
        in_specs=[
            pl.BlockSpec((1, B, 4 * RNN_H), lambda t: (t, 0, 0)),
            pl.BlockSpec((1, B, 4 * RNN_H), lambda t: (T - 1 - t, 0, 0)),
            pl.BlockSpec((RNN_H, 4 * RNN_H), lambda t: (0, 0)),
            pl.BlockSpec((RNN_H, 4 * RNN_H), lambda t: (0, 0)),
        ],
        out_specs=[
            pl.BlockSpec((B, RNN_H), lambda t: (0, 0)),
            pl.BlockSpec((B, RNN_H), lambda t: (0, 0)),
        ],
        out_shape=[
            jax.ShapeDtypeStruct((B, RNN_H), jnp.float32),
            jax.ShapeDtypeStruct((B, RNN_H), jnp.float32),
        ],
        scratch_shapes=[pltpu.VMEM((B, RNN_H), jnp.float32)] * 4,
    )(gxf, gxb, whfT, whbT)


def _head_body(hb, hf, wfa, wfb, bfc, afc, wcls, bcls, out):
    o = (jnp.dot(hb[...], wfa[...], preferred_element_type=jnp.float32)
         + jnp.dot(hf[...], wfb[...], preferred_element_type=jnp.float32)
         + bfc[...])
    o = jnp.where(o >= 0, o, afc[...] * o)
    out[...] = jnp.dot(o, wcls[...], preferred_element_type=jnp.float32) + bcls[...]


def _head(h1b, h1f, wfaT, wfbT, bfc, afc, wclsT, bcls):
    full = lambda shape: pl.BlockSpec(shape, lambda: (0,) * len(shape))
    return pl.pallas_call(
        _head_body,
        in_specs=[
            full((B, RNN_H)), full((B, RNN_H)),
            full((RNN_H, GCN_OUT)), full((RNN_H, GCN_OUT)),
            full((1, GCN_OUT)), full((1, GCN_OUT)),
            full((GCN_OUT, NUM_CLASSES)), full((1, NUM_CLASSES)),
        ],
        out_specs=full((B, NUM_CLASSES)),
        out_shape=jax.ShapeDtypeStruct((B, NUM_CLASSES), jnp.float32),
    )(h1b, h1f, wfaT, wfbT, bfc, afc, wclsT, bcls)


def _run_gcn_pair(gps, feats, eis, zeros8, ones8, zeros32, gmat):
    xs, degs, srcs, dsts = [], [], [], []
    for gp, feat, ei in zip(gps, feats, eis):
        src = ei[0].astype(jnp.int32)
        dst = ei[1].astype(jnp.int32)
        feat2 = feat.astype(jnp.int32).reshape(NROWS, CH)
        dst2 = dst.reshape(EROWS, CH)
        emb2 = jnp.transpose(gp["emb"].reshape(257, 2, 32), (1, 0, 2)).reshape(514, 32)
        x, degc = _sc_prep(feat2, emb2, dst2, zeros8, ones8)
        xs.append(x)
        degs.append(degc)
        srcs.append(src.reshape(ESROWS, CS))
        dsts.append(dst.reshape(ESROWS, CS))

    pooled = [[], []]
    for i in range(4):
        ss = [_sc_segsum(xs[g].reshape(2 * N, 32), srcs[g], dsts[g], zeros32)
              for g in range(2)]
        for g in range(2):
            gp = gps[g]
            t, mom = _k1(xs[g], ss[g], degs[g],
                         gp["Wself%d" % i].T, gp["Wneigh%d" % i].T,
                         gp["b%d" % i].reshape(1, 64), gp["a%d" % i].reshape(1, 64))
            xs[g], pi = _k2(t, mom, gp["gamma%d" % i].reshape(1, 64),
                            gp["beta%d" % i].reshape(1, 64), gmat)
            pooled[g].append(pi)
    return pooled


def kernel(params, header_feat, header_edge_index, payload_feat,
           payload_edge_index, labels):
    p = params
    zeros8 = jnp.zeros((NSTRIPE, 8), jnp.float32)
    ones8 = jnp.ones((CH, 8), jnp.float32)
    zeros32 = jnp.zeros((NSTRIPE, 32), jnp.float32)
    gmat = jnp.kron(jnp.eye(_BS // NPG, dtype=jnp.float32),
                    jnp.ones((1, NPG), jnp.float32)) * (1.0 / NPG)

    hp, pp = _run_gcn_pair((p["hg"], p["pg"]),
                           (header_feat, payload_feat),
                           (header_edge_index, payload_edge_index),
                           zeros8, ones8, zeros32, gmat)

    a1r = jnp.tile(p["f1a"], B).reshape(B * T, 1)
    a2r = jnp.tile(p["f2a"], B).reshape(B * T, 1)
    g = _filter(hp, pp,
                p["f1W1"].T, p["f1b1"].reshape(1, 256), a1r,
                p["f1W2"].T, p["f1b2"].reshape(1, 256),
                p["f2W1"].T, p["f2b1"].reshape(1, 256), a2r,
                p["f2W2"].T, p["f2b2"].reshape(1, 256))

    xs_flat = jnp.transpose(g.reshape(B, T, 512), (1, 0, 2)).reshape(B * T, 512)

    l0f, l0b = p["lstm0f"], p["lstm0b"]
    gx0f = _gates1(xs_flat, l0f["Wih"].T,
                   (l0f["bih"] + l0f["bhh"]).reshape(1, 4 * RNN_H))
    gx0b = _gates1(xs_flat, l0b["Wih"].T,
                   (l0b["bih"] + l0b["bhh"]).reshape(1, 4 * RNN_H))
    y0f, y0b = _lstm0(gx0f.reshape(T, B, 4 * RNN_H),
                      gx0b.reshape(T, B, 4 * RNN_H),
                      l0f["Whh"].T, l0b["Whh"].T)

    l1f, l1b = p["lstm1f"], p["lstm1b"]
    y0f_flat = y0f.reshape(B * T, RNN_H)
    y0b_flat = y0b.reshape(B * T, RNN_H)
    w1f = l1f["Wih"].T
    w1b = l1b["Wih"].T
    gx1f = _gates2(y0f_flat, y0b_flat, w1f[:RNN_H], w1f[RNN_H:],
                   (l1f["bih"] + l1f["bhh"]).reshape(1, 4 * RNN_H))
    gx1b = _gates2(y0f_flat, y0b_flat, w1b[:RNN_H], w1b[RNN_H:],
                   (l1b["bih"] + l1b["bhh"]).reshape(1, 4 * RNN_H))
    h1f, h1b = _lstm1(gx1f.reshape(T, B, 4 * RNN_H),
                      gx1b.reshape(T, B, 4 * RNN_H),
                      l1f["Whh"].T, l1b["Whh"].T)

    wfcT = p["Wfc"].T
    return _head(h1b, h1f, wfcT[:RNN_H], wfcT[RNN_H:],
                 p["bfc"].reshape(1, GCN_OUT), p["afc"].reshape(1, GCN_OUT),
                 p["Wcls"].T, p["bcls"].reshape(1, NUM_CLASSES))
